# Initial kernel scaffold; baseline (speedup 1.0000x reference)
import jax
import jax.numpy as jnp
from jax.experimental import pallas as pl


def kernel(img, spx):
    raise NotImplementedError("write your pallas kernel here")


# speedup vs baseline: 3.2118x; 3.2118x over previous
import functools

import jax
import jax.numpy as jnp
from jax import lax
from jax.experimental import pallas as pl
from jax.experimental.pallas import tpu as pltpu
from jax.experimental.pallas import tpu_sc as plsc

_NC, _NS, _L = 2, 16, 16
_NW = _NC * _NS
_K = 1024
_B, _C, _P = 4, 96, 384 * 384
_CB = _B * _C // _NW
_NCB = _C // _CB
_S = 2048
_NCHUNK = _P // _S


def _body(img_hbm, spx_hbm, out_hbm, idx_v, val_v, acc_v, cnt_v, sem):
    wid = lax.axis_index("s") * _NC + lax.axis_index("c")
    b = wid // _NCB
    c0 = (wid % _NCB) * _CB

    zeros = jnp.zeros((_L,), jnp.float32)
    ones = jnp.ones((_L,), jnp.float32)

    def zero_cnt(g, _):
        cnt_v[pl.ds(g * _L, _L)] = zeros
        return 0

    lax.fori_loop(0, _K // _L, zero_cnt, 0)

    def zero_acc(g, _):
        for c in range(_CB):
            acc_v[c, pl.ds(g * _L, _L)] = zeros
        return 0

    lax.fori_loop(0, _K // _L, zero_acc, 0)

    def chunk(ci, _):
        p0 = ci * _S
        pltpu.sync_copy(spx_hbm.at[b, pl.ds(p0, _S)], idx_v)
        pltpu.sync_copy(img_hbm.at[b, pl.ds(c0, _CB), pl.ds(p0, _S)], val_v)

        def group(g, _):
            idx = idx_v[pl.ds(g * _L, _L)]
            plsc.addupdate_scatter(cnt_v, [idx], ones)
            for c in range(_CB):
                v = val_v[c, pl.ds(g * _L, _L)]
                plsc.addupdate_scatter(acc_v.at[c], [idx], v)
            return 0

        lax.fori_loop(0, _S // _L, group, 0)
        return 0

    lax.fori_loop(0, _NCHUNK, chunk, 0)

    def norm(g, _):
        cnt = cnt_v[pl.ds(g * _L, _L)]
        r = 1.0 / jnp.maximum(cnt, 1.0)
        for c in range(_CB):
            acc_v[c, pl.ds(g * _L, _L)] = acc_v[c, pl.ds(g * _L, _L)] * r
        return 0

    lax.fori_loop(0, _K // _L, norm, 0)
    pltpu.sync_copy(acc_v, out_hbm.at[b, pl.ds(c0, _CB)])


@jax.jit
def kernel(img, spx):
    B, C, H, W = img.shape
    img_r = img.reshape(B, C, H * W)
    spx_r = spx.reshape(B, H * W)
    mesh = plsc.VectorSubcoreMesh(
        core_axis_name="c", subcore_axis_name="s",
        num_cores=_NC, num_subcores=_NS,
    )
    f = pl.kernel(
        _body,
        out_type=jax.ShapeDtypeStruct((_B, _C, _K), jnp.float32),
        mesh=mesh,
        compiler_params=pltpu.CompilerParams(
            use_tc_tiling_on_sc=False, needs_layout_passes=False),
        scratch_types=[
            pltpu.VMEM((_S,), jnp.int32),
            pltpu.VMEM((_CB, _S), jnp.float32),
            pltpu.VMEM((_CB, _K), jnp.float32),
            pltpu.VMEM((_K,), jnp.float32),
            pltpu.SemaphoreType.DMA,
        ],
    )
    return f(img_r, spx_r)


# speedup vs baseline: 3.8936x; 1.2123x over previous
import functools

import jax
import jax.numpy as jnp
from jax import lax
from jax.experimental import pallas as pl
from jax.experimental.pallas import tpu as pltpu
from jax.experimental.pallas import tpu_sc as plsc

_NC, _NS, _L = 2, 16, 16
_NW = _NC * _NS
_K = 1024
_B, _C, _P = 4, 96, 384 * 384
_CB = _B * _C // _NW
_NCB = _C // _CB
_S = 2048
_NCHUNK = _P // _S


def _body(img_hbm, spx_hbm, out_hbm, idx_v, val_v, acc_v, cnt_v, sem0, sem1):
    wid = lax.axis_index("s") * _NC + lax.axis_index("c")
    b = wid // _NCB
    c0 = (wid % _NCB) * _CB
    sems = (sem0, sem1)

    zeros = jnp.zeros((_L,), jnp.float32)
    ones = jnp.ones((_L,), jnp.float32)

    def zero_cnt(g, _):
        cnt_v[pl.ds(g * _L, _L)] = zeros
        return 0

    lax.fori_loop(0, _K // _L, zero_cnt, 0)

    def zero_acc(g, _):
        for c in range(_CB):
            acc_v[c, pl.ds(g * _L, _L)] = zeros
        return 0

    lax.fori_loop(0, _K // _L, zero_acc, 0)

    def copies(ci, s):
        p0 = ci * _S
        return (
            pltpu.make_async_copy(
                spx_hbm.at[b, pl.ds(p0, _S)], idx_v.at[s], sems[s]),
            pltpu.make_async_copy(
                img_hbm.at[b, pl.ds(c0, _CB), pl.ds(p0, _S)], val_v.at[s],
                sems[s]),
        )

    def issue(ci, s):
        for d in copies(ci, s):
            d.start()

    issue(0, 0)

    @pl.loop(0, _NCHUNK, step=2)
    def chunk(g):
        for s in range(2):
            ci = g + s

            @pl.when(ci + 1 < _NCHUNK)
            def _():
                issue(ci + 1, 1 - s)

            for d in copies(ci, s):
                d.wait()

            @pl.loop(0, _S // _L, unroll=4)
            def group(gg):
                idx = idx_v[s, pl.ds(gg * _L, _L)]
                plsc.addupdate_scatter(cnt_v, [idx], ones)
                for c in range(_CB):
                    v = val_v[s, c, pl.ds(gg * _L, _L)]
                    plsc.addupdate_scatter(acc_v.at[c], [idx], v)

    def norm(g, _):
        cnt = cnt_v[pl.ds(g * _L, _L)]
        r = 1.0 / jnp.maximum(cnt, 1.0)
        for c in range(_CB):
            acc_v[c, pl.ds(g * _L, _L)] = acc_v[c, pl.ds(g * _L, _L)] * r
        return 0

    lax.fori_loop(0, _K // _L, norm, 0)
    pltpu.sync_copy(acc_v, out_hbm.at[b, pl.ds(c0, _CB)])


@jax.jit
def kernel(img, spx):
    B, C, H, W = img.shape
    img_r = img.reshape(B, C, H * W)
    spx_r = spx.reshape(B, H * W)
    mesh = plsc.VectorSubcoreMesh(
        core_axis_name="c", subcore_axis_name="s",
        num_cores=_NC, num_subcores=_NS,
    )
    f = pl.kernel(
        _body,
        out_type=jax.ShapeDtypeStruct((_B, _C, _K), jnp.float32),
        mesh=mesh,
        compiler_params=pltpu.CompilerParams(
            use_tc_tiling_on_sc=False, needs_layout_passes=False),
        scratch_types=[
            pltpu.VMEM((2, _S), jnp.int32),
            pltpu.VMEM((2, _CB, _S), jnp.float32),
            pltpu.VMEM((_CB, _K), jnp.float32),
            pltpu.VMEM((_K,), jnp.float32),
            pltpu.SemaphoreType.DMA,
            pltpu.SemaphoreType.DMA,
        ],
    )
    return f(img_r, spx_r)


# speedup vs baseline: 3.8963x; 1.0007x over previous
import functools

import jax
import jax.numpy as jnp
from jax import lax
from jax.experimental import pallas as pl
from jax.experimental.pallas import tpu as pltpu
from jax.experimental.pallas import tpu_sc as plsc

_NC, _NS, _L = 2, 16, 16
_NW = _NC * _NS
_K = 1024
_B, _C, _P = 4, 96, 384 * 384
_CB = _B * _C // _NW
_NCB = _C // _CB
_S = 2048
_NCHUNK = _P // _S


def _body(img_hbm, spx_hbm, out_hbm, idx_v, val_v, out_v, cnt_v, sem0, sem1,
          *accs):
    wid = lax.axis_index("s") * _NC + lax.axis_index("c")
    b = wid // _NCB
    c0 = (wid % _NCB) * _CB
    sems = (sem0, sem1)

    zeros = jnp.zeros((_L,), jnp.float32)
    ones = jnp.ones((_L,), jnp.float32)

    def zero(g, _):
        cnt_v[pl.ds(g * _L, _L)] = zeros
        for c in range(_CB):
            accs[c][pl.ds(g * _L, _L)] = zeros
        return 0

    lax.fori_loop(0, _K // _L, zero, 0)

    def copies(ci, s):
        p0 = ci * _S
        return (
            pltpu.make_async_copy(
                spx_hbm.at[b, pl.ds(p0, _S)], idx_v.at[s], sems[s]),
            pltpu.make_async_copy(
                img_hbm.at[b, pl.ds(c0, _CB), pl.ds(p0, _S)], val_v.at[s],
                sems[s]),
        )

    def issue(ci, s):
        for d in copies(ci, s):
            d.start()

    issue(0, 0)

    @pl.loop(0, _NCHUNK, step=2)
    def chunk(g):
        for s in range(2):
            ci = g + s

            @pl.when(ci + 1 < _NCHUNK)
            def _():
                issue(ci + 1, 1 - s)

            for d in copies(ci, s):
                d.wait()

            @pl.loop(0, _S // _L, unroll=4)
            def group(gg):
                idx = idx_v[s, pl.ds(gg * _L, _L)]
                plsc.addupdate_scatter(cnt_v, [idx], ones)
                for c in range(_CB):
                    v = val_v[s, c, pl.ds(gg * _L, _L)]
                    plsc.addupdate_scatter(accs[c], [idx], v)

    def norm(g, _):
        cnt = cnt_v[pl.ds(g * _L, _L)]
        r = 1.0 / jnp.maximum(cnt, 1.0)
        for c in range(_CB):
            out_v[c, pl.ds(g * _L, _L)] = accs[c][pl.ds(g * _L, _L)] * r
        return 0

    lax.fori_loop(0, _K // _L, norm, 0)
    pltpu.sync_copy(out_v, out_hbm.at[b, pl.ds(c0, _CB)])


@jax.jit
def kernel(img, spx):
    B, C, H, W = img.shape
    img_r = img.reshape(B, C, H * W)
    spx_r = spx.reshape(B, H * W)
    mesh = plsc.VectorSubcoreMesh(
        core_axis_name="c", subcore_axis_name="s",
        num_cores=_NC, num_subcores=_NS,
    )
    f = pl.kernel(
        _body,
        out_type=jax.ShapeDtypeStruct((_B, _C, _K), jnp.float32),
        mesh=mesh,
        compiler_params=pltpu.CompilerParams(
            use_tc_tiling_on_sc=False, needs_layout_passes=False),
        scratch_types=[
            pltpu.VMEM((2, _S), jnp.int32),
            pltpu.VMEM((2, _CB, _S), jnp.float32),
            pltpu.VMEM((_CB, _K), jnp.float32),
            pltpu.VMEM((_K,), jnp.float32),
            pltpu.SemaphoreType.DMA,
            pltpu.SemaphoreType.DMA,
        ] + [pltpu.VMEM((_K,), jnp.float32) for _ in range(_CB)],
    )
    return f(img_r, spx_r)


# speedup vs baseline: 6.1092x; 1.5679x over previous
import functools

import jax
import jax.numpy as jnp
from jax import lax
from jax.experimental import pallas as pl
from jax.experimental.pallas import tpu as pltpu
from jax.experimental.pallas import tpu_sc as plsc

_NC, _NS, _L = 2, 16, 16
_NW = _NC * _NS
_K = 1024
_B, _C, _P = 4, 96, 384 * 384
_CB = _B * _C // _NW
_NCB = _C // _CB
_S = 2048
_NCHUNK = _P // _S


def _body(img_hbm, spx_hbm, out_hbm, idx_v, val_v, out_v, cnt_v, sem0, sem1,
          *accs):
    wid = lax.axis_index("s") * _NC + lax.axis_index("c")
    b = wid // _NCB
    c0 = (wid % _NCB) * _CB
    sems = (sem0, sem1)

    zeros = jnp.zeros((_L,), jnp.float32)
    ones = jnp.ones((_L,), jnp.float32)

    def zero(g, _):
        cnt_v[pl.ds(g * _L, _L)] = zeros
        for c in range(_CB):
            accs[c][pl.ds(g * _L, _L)] = zeros
        return 0

    lax.fori_loop(0, _K // _L, zero, 0)

    def copies(ci, s):
        p0 = ci * _S
        return (
            pltpu.make_async_copy(
                spx_hbm.at[b, pl.ds(p0, _S)], idx_v.at[s], sems[s]),
            pltpu.make_async_copy(
                img_hbm.at[b, pl.ds(c0, _CB), pl.ds(p0, _S)], val_v.at[s],
                sems[s]),
        )

    def issue(ci, s):
        for d in copies(ci, s):
            d.start()

    issue(0, 0)

    @pl.loop(0, _NCHUNK, step=2)
    def chunk(g):
        for s in range(2):
            ci = g + s

            @pl.when(ci + 1 < _NCHUNK)
            def _():
                issue(ci + 1, 1 - s)

            for d in copies(ci, s):
                d.wait()

            @plsc.parallel_loop(0, _S // _L, unroll=2)
            def group(gg):
                idx = idx_v[s, pl.ds(gg * _L, _L)]
                vals = [val_v[s, c, pl.ds(gg * _L, _L)] for c in range(_CB)]
                plsc.addupdate_scatter(cnt_v, [idx], ones)
                for c in range(_CB):
                    plsc.addupdate_scatter(accs[c], [idx], vals[c])

    def norm(g, _):
        cnt = cnt_v[pl.ds(g * _L, _L)]
        r = 1.0 / jnp.maximum(cnt, 1.0)
        for c in range(_CB):
            out_v[c, pl.ds(g * _L, _L)] = accs[c][pl.ds(g * _L, _L)] * r
        return 0

    lax.fori_loop(0, _K // _L, norm, 0)
    pltpu.sync_copy(out_v, out_hbm.at[b, pl.ds(c0, _CB)])


@jax.jit
def kernel(img, spx):
    B, C, H, W = img.shape
    img_r = img.reshape(B, C, H * W)
    spx_r = spx.reshape(B, H * W)
    mesh = plsc.VectorSubcoreMesh(
        core_axis_name="c", subcore_axis_name="s",
        num_cores=_NC, num_subcores=_NS,
    )
    f = pl.kernel(
        _body,
        out_type=jax.ShapeDtypeStruct((_B, _C, _K), jnp.float32),
        mesh=mesh,
        compiler_params=pltpu.CompilerParams(
            use_tc_tiling_on_sc=False, needs_layout_passes=False),
        scratch_types=[
            pltpu.VMEM((2, _S), jnp.int32),
            pltpu.VMEM((2, _CB, _S), jnp.float32),
            pltpu.VMEM((_CB, _K), jnp.float32),
            pltpu.VMEM((_K,), jnp.float32),
            pltpu.SemaphoreType.DMA,
            pltpu.SemaphoreType.DMA,
        ] + [pltpu.VMEM((_K,), jnp.float32) for _ in range(_CB)],
    )
    return f(img_r, spx_r)


# speedup vs baseline: 6.1463x; 1.0061x over previous
import functools

import jax
import jax.numpy as jnp
from jax import lax
from jax.experimental import pallas as pl
from jax.experimental.pallas import tpu as pltpu
from jax.experimental.pallas import tpu_sc as plsc

_NC, _NS, _L = 2, 16, 16
_NW = _NC * _NS
_K = 1024
_B, _C, _P = 4, 96, 384 * 384
_CB = _B * _C // _NW
_NCB = _C // _CB
_S = 2048
_NCHUNK = _P // _S


def _body(img_hbm, spx_hbm, out_hbm, idx_v, val_v, out_v, cnt_v, sem0, sem1,
          *accs):
    wid = lax.axis_index("s") * _NC + lax.axis_index("c")
    b = wid // _NCB
    c0 = (wid % _NCB) * _CB
    sems = (sem0, sem1)

    zeros = jnp.zeros((_L,), jnp.float32)
    ones = jnp.ones((_L,), jnp.float32)

    def zero(g, _):
        cnt_v[pl.ds(g * _L, _L)] = zeros
        for c in range(_CB):
            accs[c][pl.ds(g * _L, _L)] = zeros
        return 0

    lax.fori_loop(0, _K // _L, zero, 0)

    def copies(ci, s):
        p0 = ci * _S
        return (
            pltpu.make_async_copy(
                spx_hbm.at[b, pl.ds(p0, _S)], idx_v.at[s], sems[s]),
            pltpu.make_async_copy(
                img_hbm.at[b, pl.ds(c0, _CB), pl.ds(p0, _S)], val_v.at[s],
                sems[s]),
        )

    def issue(ci, s):
        for d in copies(ci, s):
            d.start()

    issue(0, 0)

    @pl.loop(0, _NCHUNK, step=2)
    def chunk(g):
        for s in range(2):
            ci = g + s

            @pl.when(ci + 1 < _NCHUNK)
            def _():
                issue(ci + 1, 1 - s)

            for d in copies(ci, s):
                d.wait()

            @plsc.parallel_loop(0, _S // _L, unroll=4)
            def group(gg):
                idx = idx_v[s, pl.ds(gg * _L, _L)]
                vals = [val_v[s, c, pl.ds(gg * _L, _L)] for c in range(_CB)]
                plsc.addupdate_scatter(cnt_v, [idx], ones)
                for c in range(_CB):
                    plsc.addupdate_scatter(accs[c], [idx], vals[c])

    def norm(g, _):
        cnt = cnt_v[pl.ds(g * _L, _L)]
        r = 1.0 / jnp.maximum(cnt, 1.0)
        for c in range(_CB):
            out_v[c, pl.ds(g * _L, _L)] = accs[c][pl.ds(g * _L, _L)] * r
        return 0

    lax.fori_loop(0, _K // _L, norm, 0)
    pltpu.sync_copy(out_v, out_hbm.at[b, pl.ds(c0, _CB)])


@jax.jit
def kernel(img, spx):
    B, C, H, W = img.shape
    img_r = img.reshape(B, C, H * W)
    spx_r = spx.reshape(B, H * W)
    mesh = plsc.VectorSubcoreMesh(
        core_axis_name="c", subcore_axis_name="s",
        num_cores=_NC, num_subcores=_NS,
    )
    f = pl.kernel(
        _body,
        out_type=jax.ShapeDtypeStruct((_B, _C, _K), jnp.float32),
        mesh=mesh,
        compiler_params=pltpu.CompilerParams(
            use_tc_tiling_on_sc=False, needs_layout_passes=False),
        scratch_types=[
            pltpu.VMEM((2, _S), jnp.int32),
            pltpu.VMEM((2, _CB, _S), jnp.float32),
            pltpu.VMEM((_CB, _K), jnp.float32),
            pltpu.VMEM((_K,), jnp.float32),
            pltpu.SemaphoreType.DMA,
            pltpu.SemaphoreType.DMA,
        ] + [pltpu.VMEM((_K,), jnp.float32) for _ in range(_CB)],
    )
    return f(img_r, spx_r)


# speedup vs baseline: 6.1481x; 1.0003x over previous
import functools

import jax
import jax.numpy as jnp
from jax import lax
from jax.experimental import pallas as pl
from jax.experimental.pallas import tpu as pltpu
from jax.experimental.pallas import tpu_sc as plsc

_NC, _NS, _L = 2, 16, 16
_NW = _NC * _NS
_K = 1024
_B, _C, _P = 4, 96, 384 * 384
_CB = _B * _C // _NW
_NCB = _C // _CB
_S = 2048
_NCHUNK = _P // _S


def _body(img_hbm, spx_hbm, out_hbm, idx_v, val_v, out_v, cnt_v, sem0, sem1,
          *accs):
    wid = lax.axis_index("s") * _NC + lax.axis_index("c")
    b = wid // _NCB
    c0 = (wid % _NCB) * _CB
    sems = (sem0, sem1)

    zeros = jnp.zeros((_L,), jnp.float32)
    ones = jnp.ones((_L,), jnp.float32)

    def zero(g, _):
        cnt_v[pl.ds(g * _L, _L)] = zeros
        for c in range(_CB):
            accs[c][pl.ds(g * _L, _L)] = zeros
        return 0

    lax.fori_loop(0, _K // _L, zero, 0)

    def copies(ci, s):
        p0 = ci * _S
        return (
            pltpu.make_async_copy(
                spx_hbm.at[b, pl.ds(p0, _S)], idx_v.at[s], sems[s]),
            pltpu.make_async_copy(
                img_hbm.at[b, pl.ds(c0, _CB), pl.ds(p0, _S)], val_v.at[s],
                sems[s]),
        )

    def issue(ci, s):
        for d in copies(ci, s):
            d.start()

    issue(0, 0)

    @pl.loop(0, _NCHUNK, step=2)
    def chunk(g):
        for s in range(2):
            ci = g + s

            @pl.when(ci + 1 < _NCHUNK)
            def _():
                issue(ci + 1, 1 - s)

            for d in copies(ci, s):
                d.wait()

            def load(gg):
                idx = idx_v[s, pl.ds(gg * _L, _L)]
                vals = [val_v[s, c, pl.ds(gg * _L, _L)] for c in range(_CB)]
                return (idx, vals)

            def scat(idx, vals):
                plsc.addupdate_scatter(cnt_v, [idx], ones)
                for c in range(_CB):
                    plsc.addupdate_scatter(accs[c], [idx], vals[c])

            @plsc.parallel_loop(0, _S // _L - 1, unroll=4, carry=load(0))
            def group(gg, carry):
                nxt = load(gg + 1)
                scat(*carry)
                return nxt

            scat(*group)

    def norm(g, _):
        cnt = cnt_v[pl.ds(g * _L, _L)]
        r = 1.0 / jnp.maximum(cnt, 1.0)
        for c in range(_CB):
            out_v[c, pl.ds(g * _L, _L)] = accs[c][pl.ds(g * _L, _L)] * r
        return 0

    lax.fori_loop(0, _K // _L, norm, 0)
    pltpu.sync_copy(out_v, out_hbm.at[b, pl.ds(c0, _CB)])


@jax.jit
def kernel(img, spx):
    B, C, H, W = img.shape
    img_r = img.reshape(B, C, H * W)
    spx_r = spx.reshape(B, H * W)
    mesh = plsc.VectorSubcoreMesh(
        core_axis_name="c", subcore_axis_name="s",
        num_cores=_NC, num_subcores=_NS,
    )
    f = pl.kernel(
        _body,
        out_type=jax.ShapeDtypeStruct((_B, _C, _K), jnp.float32),
        mesh=mesh,
        compiler_params=pltpu.CompilerParams(
            use_tc_tiling_on_sc=False, needs_layout_passes=False),
        scratch_types=[
            pltpu.VMEM((2, _S), jnp.int32),
            pltpu.VMEM((2, _CB, _S), jnp.float32),
            pltpu.VMEM((_CB, _K), jnp.float32),
            pltpu.VMEM((_K,), jnp.float32),
            pltpu.SemaphoreType.DMA,
            pltpu.SemaphoreType.DMA,
        ] + [pltpu.VMEM((_K,), jnp.float32) for _ in range(_CB)],
    )
    return f(img_r, spx_r)
